# baseline (speedup 1.0000x reference)
import functools

import jax
import jax.numpy as jnp
from jax import lax
from jax.experimental import pallas as pl
from jax.experimental.pallas import tpu as pltpu
from jax.experimental.pallas import tpu_sc as plsc

_ROWS = 1000000
_BATCH = 16384
_FIELDS = 26
_DIM = 64
_PAD = 128
_L = 16

_NC = 2
_NS = 16
_NW = _NC * _NS
_BPW = _BATCH // _NW
_CHUNK = 128
_CPF = _BPW // _CHUNK
_NCH = _FIELDS * _CPF
_NBUF = 4


def _embed_body(tbl_hbm, idx_hbm, out_hbm, idx_v, rows_v, trans_v, gsem, osem):
    wid = lax.axis_index("s") * _NC + lax.axis_index("c")
    base = wid * _BPW

    pltpu.sync_copy(idx_hbm.at[:, pl.ds(base, _BPW)], idx_v)

    def start_gather(k, slot):
        f = k // _CPF
        c = lax.rem(k, _CPF)
        pltpu.make_async_copy(
            tbl_hbm.at[idx_v.at[f, pl.ds(c * _CHUNK, _CHUNK)]],
            rows_v.at[slot],
            gsem.at[slot],
        ).start()

    def out_copy(k, slot):
        f = k // _CPF
        c = lax.rem(k, _CPF)
        return pltpu.make_async_copy(
            trans_v.at[slot],
            out_hbm.at[f, :, pl.ds(base + c * _CHUNK, _CHUNK)],
            osem.at[slot],
        )

    def transpose_chunk(slot):
        def body_e(e):
            ev = jnp.broadcast_to(e, (_L,)).astype(jnp.int32)
            for g in range(_CHUNK // _L):
                bi = lax.iota(jnp.int32, _L) + (g * _L)
                v = plsc.load_gather(rows_v.at[slot], [bi, ev])
                trans_v[slot, e, pl.ds(g * _L, _L)] = v

        pl.loop(0, _DIM)(body_e)

    for b in range(_NBUF):
        start_gather(b, b)

    def outer(k0):
        for b in range(_NBUF):
            k = k0 + b
            pltpu.make_async_copy(
                tbl_hbm.at[idx_v.at[0, pl.ds(0, _CHUNK)]],
                rows_v.at[b],
                gsem.at[b],
            ).wait()

            @pl.when(k >= _NBUF)
            def _():
                out_copy(k, b).wait()

            transpose_chunk(b)
            out_copy(k, b).start()

            @pl.when(k + _NBUF < _NCH)
            def _():
                start_gather(k + _NBUF, b)

    pl.loop(0, _NCH, step=_NBUF)(outer)

    for b in range(_NBUF):
        out_copy(_NCH - _NBUF + b, b).wait()


@functools.partial(
    pl.kernel,
    mesh=plsc.VectorSubcoreMesh(core_axis_name="c", subcore_axis_name="s"),
    out_type=jax.ShapeDtypeStruct((_FIELDS, _DIM, _BATCH), jnp.float32),
    scratch_types=[
        pltpu.VMEM((_FIELDS, _BPW), jnp.int32),
        pltpu.VMEM((_NBUF, _CHUNK, _PAD), jnp.float32),
        pltpu.VMEM((_NBUF, _DIM, _CHUNK), jnp.float32),
        pltpu.SemaphoreType.DMA((_NBUF,)),
        pltpu.SemaphoreType.DMA((_NBUF,)),
    ],
    compiler_params=pltpu.CompilerParams(
        use_tc_tiling_on_sc=True, needs_layout_passes=False
    ),
)
def _embed_call(tbl_hbm, idx_hbm, out_hbm, idx_v, rows_v, trans_v, gsem, osem):
    _embed_body(tbl_hbm, idx_hbm, out_hbm, idx_v, rows_v, trans_v, gsem, osem)


def kernel(input, weight):
    wpad = jnp.pad(weight, ((0, 0), (0, _PAD - _DIM)))
    idx_t = input.astype(jnp.int32).T
    out = _embed_call(wpad, idx_t)
    return jnp.transpose(out, (2, 0, 1))


# speedup vs baseline: 1.5022x; 1.5022x over previous
import functools

import jax
import jax.numpy as jnp
from jax import lax
from jax.experimental import pallas as pl
from jax.experimental.pallas import tpu as pltpu
from jax.experimental.pallas import tpu_sc as plsc

_BATCH = 16384
_FIELDS = 26
_DIM = 64
_PAD = 128

_NC = 2
_NS = 16
_NW = _NC * _NS
_BPW = _BATCH // _NW
_CHUNK = 64
_CPF = _BPW // _CHUNK
_NCH = _FIELDS * _CPF
_NBUF = 8


def _embed_body(tbl_hbm, idx_hbm, out_hbm, idx_v, rows_v, gsem):
    wid = lax.axis_index("s") * _NC + lax.axis_index("c")
    base = wid * _BPW

    pltpu.sync_copy(idx_hbm.at[:, pl.ds(base, _BPW)], idx_v)

    def start_gather(k, slot):
        f = k // _CPF
        c = lax.rem(k, _CPF)
        pltpu.make_async_copy(
            tbl_hbm.at[idx_v.at[f, pl.ds(c * _CHUNK, _CHUNK)]],
            rows_v.at[slot],
            gsem.at[slot],
        ).start()

    for b in range(_NBUF):
        start_gather(b, b)

    def outer(k0):
        for b in range(_NBUF):
            k = k0 + b
            f = k // _CPF
            c = lax.rem(k, _CPF)
            pltpu.make_async_copy(
                tbl_hbm.at[idx_v.at[f, pl.ds(c * _CHUNK, _CHUNK)]],
                rows_v.at[b],
                gsem.at[b],
            ).wait()
            pltpu.sync_copy(
                rows_v.at[b],
                out_hbm.at[pl.ds(base + c * _CHUNK, _CHUNK), f],
            )

            @pl.when(k + _NBUF < _NCH)
            def _():
                start_gather(k + _NBUF, b)

    pl.loop(0, _NCH, step=_NBUF)(outer)


@functools.partial(
    pl.kernel,
    mesh=plsc.VectorSubcoreMesh(core_axis_name="c", subcore_axis_name="s"),
    out_type=jax.ShapeDtypeStruct((_BATCH, _FIELDS, _PAD), jnp.float32),
    scratch_types=[
        pltpu.VMEM((_FIELDS, _BPW), jnp.int32),
        pltpu.VMEM((_NBUF, _CHUNK, _PAD), jnp.float32),
        pltpu.SemaphoreType.DMA((_NBUF,)),
    ],
    compiler_params=pltpu.CompilerParams(use_tc_tiling_on_sc=True),
)
def _embed_call(tbl_hbm, idx_hbm, out_hbm, idx_v, rows_v, gsem):
    _embed_body(tbl_hbm, idx_hbm, out_hbm, idx_v, rows_v, gsem)


def kernel(input, weight):
    wpad = jnp.pad(weight, ((0, 0), (0, _PAD - _DIM)))
    idx_t = input.astype(jnp.int32).T
    return _embed_call(wpad, idx_t)[:, :, :_DIM]


# speedup vs baseline: 1.7024x; 1.1333x over previous
import functools

import jax
import jax.numpy as jnp
from jax import lax
from jax.experimental import pallas as pl
from jax.experimental.pallas import tpu as pltpu
from jax.experimental.pallas import tpu_sc as plsc

_ROWS = 1000000
_BATCH = 16384
_FIELDS = 26
_DIM = 64

_NC = 2
_NS = 16
_NW = _NC * _NS
_BPW = _BATCH // _NW
_CHUNK = 128
_CPF = _BPW // _CHUNK
_NCH = _FIELDS * _CPF
_NBUF = 4
_UNROLL = 4


def _embed_body(tbl_hbm, idx_hbm, out_hbm, idx_v, rows_v, gsem, osem):
    wid = lax.axis_index("s") * _NC + lax.axis_index("c")
    base = wid * _BPW

    pltpu.sync_copy(idx_hbm.at[:, pl.ds(base, _BPW)], idx_v)

    def issue_chunk(k, slot):
        f = k // _CPF
        jbase = lax.rem(k, _CPF) * _CHUNK

        def issue(j0):
            iv = idx_v[f, pl.ds(jbase + j0, 16)]
            for u in range(16):
                pltpu.make_async_copy(
                    tbl_hbm.at[iv[u]], rows_v.at[slot, j0 + u], gsem.at[slot]
                ).start()

        pl.loop(0, _CHUNK, step=16)(issue)

    def drain_chunk(slot):
        pltpu.make_async_copy(
            tbl_hbm.at[pl.ds(0, _CHUNK)], rows_v.at[slot], gsem.at[slot]
        ).wait()

    def out_copy(k, slot):
        f = k // _CPF
        c = lax.rem(k, _CPF)
        return pltpu.make_async_copy(
            rows_v.at[slot],
            out_hbm.at[pl.ds(base + c * _CHUNK, _CHUNK), f],
            osem.at[slot],
        )

    for b in range(_NBUF):
        issue_chunk(b, b)

    def outer(k0):
        for b in range(_NBUF):
            k = k0 + b
            drain_chunk(b)
            out_copy(k, b).start()

            @pl.when(k + _NBUF < _NCH)
            def _():
                out_copy(k, b).wait()
                issue_chunk(k + _NBUF, b)

            @pl.when(k + _NBUF >= _NCH)
            def _():
                out_copy(k, b).wait()

    pl.loop(0, _NCH, step=_NBUF)(outer)


@functools.partial(
    pl.kernel,
    mesh=plsc.VectorSubcoreMesh(core_axis_name="c", subcore_axis_name="s"),
    out_type=jax.ShapeDtypeStruct((_BATCH, _FIELDS, _DIM), jnp.float32),
    scratch_types=[
        pltpu.VMEM((_FIELDS, _BPW), jnp.int32),
        pltpu.VMEM((_NBUF, _CHUNK, _DIM), jnp.float32),
        pltpu.SemaphoreType.DMA((_NBUF,)),
        pltpu.SemaphoreType.DMA((_NBUF,)),
    ],
    compiler_params=pltpu.CompilerParams(use_tc_tiling_on_sc=True),
)
def _embed_call(tbl_hbm, idx_hbm, out_hbm, idx_v, rows_v, gsem, osem):
    _embed_body(tbl_hbm, idx_hbm, out_hbm, idx_v, rows_v, gsem, osem)


def kernel(input, weight):
    idx_t = input.astype(jnp.int32).T
    return _embed_call(weight, idx_t)


# speedup vs baseline: 2.0707x; 1.2164x over previous
import functools

import jax
import jax.numpy as jnp
from jax import lax
from jax.experimental import pallas as pl
from jax.experimental.pallas import tpu as pltpu
from jax.experimental.pallas import tpu_sc as plsc

_ROWS = 1000000
_BATCH = 16384
_FIELDS = 26
_DIM = 64

_NC = 2
_NS = 16
_NW = _NC * _NS
_BPW = _BATCH // _NW
_CHUNK = 128
_CPF = _BPW // _CHUNK
_NCH = _FIELDS * _CPF
_NBUF = 4
_UNROLL = 4


def _embed_body(tbl_hbm, idx_hbm, out_hbm, idx_v, rows_v, gsem, osem):
    wid = lax.axis_index("s") * _NC + lax.axis_index("c")
    base = wid * _BPW

    pltpu.sync_copy(idx_hbm.at[:, pl.ds(base, _BPW)], idx_v)

    def issue_chunk(k, slot):
        f = k // _CPF
        jbase = lax.rem(k, _CPF) * _CHUNK

        def issue(j0):
            iv = idx_v[f, pl.ds(jbase + j0, 16)]
            for u in range(16):
                pltpu.make_async_copy(
                    tbl_hbm.at[iv[u]], rows_v.at[slot, j0 + u], gsem.at[slot]
                ).start()

        pl.loop(0, _CHUNK, step=16)(issue)

    def drain_chunk(slot):
        pltpu.make_async_copy(
            tbl_hbm.at[pl.ds(0, _CHUNK)], rows_v.at[slot], gsem.at[slot]
        ).wait()

    def out_copy(k, slot):
        f = k // _CPF
        c = lax.rem(k, _CPF)
        return pltpu.make_async_copy(
            rows_v.at[slot],
            out_hbm.at[pl.ds(base + c * _CHUNK, _CHUNK), f],
            osem.at[slot],
        )

    for b in range(_NBUF):
        issue_chunk(b, b)

    def outer(k0):
        for b in range(_NBUF):
            k = k0 + b
            drain_chunk(b)
            out_copy(k, b).start()

            @pl.when(k + _NBUF < _NCH)
            def _():
                out_copy(k, b).wait()
                issue_chunk(k + _NBUF, b)

            @pl.when(k + _NBUF >= _NCH)
            def _():
                out_copy(k, b).wait()

    pl.loop(0, _NCH, step=_NBUF)(outer)


@functools.partial(
    pl.kernel,
    mesh=plsc.VectorSubcoreMesh(core_axis_name="c", subcore_axis_name="s"),
    out_type=jax.ShapeDtypeStruct((_BATCH, _FIELDS, _DIM), jnp.float32),
    scratch_types=[
        pltpu.VMEM((_FIELDS, _BPW), jnp.int32),
        pltpu.VMEM((_NBUF, _CHUNK, _DIM), jnp.float32),
        pltpu.SemaphoreType.DMA((_NBUF,)),
        pltpu.SemaphoreType.DMA((_NBUF,)),
    ],
    compiler_params=pltpu.CompilerParams(use_tc_tiling_on_sc=True),
)
def _embed_call(tbl_hbm, idx_hbm, out_hbm, idx_v, rows_v, gsem, osem):
    _embed_body(tbl_hbm, idx_hbm, out_hbm, idx_v, rows_v, gsem, osem)


def kernel(input, weight):
    wt = jax.lax.optimization_barrier(weight.T)
    idx_t = input.astype(jnp.int32).T
    return _embed_call(wt.T, idx_t)


# speedup vs baseline: 2.4915x; 1.2032x over previous
import functools

import jax
import jax.numpy as jnp
from jax import lax
from jax.experimental import pallas as pl
from jax.experimental.pallas import tpu as pltpu
from jax.experimental.pallas import tpu_sc as plsc

_ROWS = 1000000
_BATCH = 16384
_FIELDS = 26
_DIM = 64

_NC = 2
_NS = 16
_NW = _NC * _NS
_BPW = _BATCH // _NW
_CHUNK = 128
_CPF = _BPW // _CHUNK
_NCH = _FIELDS * _CPF
_NBUF = 4
_UNROLL = 4


def _embed_body(tbl_hbm, idx_hbm, out_hbm, idx_v, rows_v, gsem, osem):
    wid = lax.axis_index("s") * _NC + lax.axis_index("c")
    base = wid * _BPW

    pltpu.sync_copy(idx_hbm.at[:, pl.ds(base, _BPW)], idx_v)

    def issue_chunk(k, slot):
        f = k // _CPF
        jbase = lax.rem(k, _CPF) * _CHUNK

        def issue(j0):
            iv = idx_v[f, pl.ds(jbase + j0, 16)]
            for u in range(16):
                pltpu.make_async_copy(
                    tbl_hbm.at[iv[u]], rows_v.at[slot, j0 + u], gsem.at[slot]
                ).start()

        pl.loop(0, _CHUNK, step=16)(issue)

    def drain_chunk(slot):
        pltpu.make_async_copy(
            tbl_hbm.at[pl.ds(0, _CHUNK)], rows_v.at[slot], gsem.at[slot]
        ).wait()

    def out_copy(k, slot):
        f = k // _CPF
        c = lax.rem(k, _CPF)
        return pltpu.make_async_copy(
            rows_v.at[slot],
            out_hbm.at[pl.ds(base + c * _CHUNK, _CHUNK), f],
            osem.at[slot],
        )

    for b in range(_NBUF):
        issue_chunk(b, b)

    def outer(k0):
        for b in range(_NBUF):
            k = k0 + b
            drain_chunk(b)
            out_copy(k, b).start()

            @pl.when(k + _NBUF < _NCH)
            def _():
                out_copy(k, b).wait()
                issue_chunk(k + _NBUF, b)

            @pl.when(k + _NBUF >= _NCH)
            def _():
                out_copy(k, b).wait()

    pl.loop(0, _NCH, step=_NBUF)(outer)


@functools.partial(
    pl.kernel,
    mesh=plsc.VectorSubcoreMesh(core_axis_name="c", subcore_axis_name="s"),
    out_type=jax.ShapeDtypeStruct((_BATCH, _FIELDS, _DIM), jnp.float32),
    scratch_types=[
        pltpu.VMEM((_FIELDS, _BPW), jnp.int32),
        pltpu.VMEM((_NBUF, _CHUNK, _DIM), jnp.float32),
        pltpu.SemaphoreType.DMA((_NBUF,)),
        pltpu.SemaphoreType.DMA((_NBUF,)),
    ],
    compiler_params=pltpu.CompilerParams(use_tc_tiling_on_sc=True),
)
def _embed_call(tbl_hbm, idx_hbm, out_hbm, idx_v, rows_v, gsem, osem):
    _embed_body(tbl_hbm, idx_hbm, out_hbm, idx_v, rows_v, gsem, osem)


def kernel(input, weight):
    wt = jax.lax.optimization_barrier(weight.T)
    idx_t = input.astype(jnp.int32).T
    out = _embed_call(wt.T, idx_t)
    out_b = jax.lax.optimization_barrier(jnp.transpose(out, (1, 2, 0)))
    return jnp.transpose(out_b, (2, 0, 1))
